# baseline (speedup 1.0000x reference)
import functools

import jax
import jax.numpy as jnp
from jax import lax
from jax.experimental import pallas as pl
from jax.experimental.pallas import tpu as pltpu
from jax.experimental.pallas import tpu_sc as plsc

_NUM = 16
_NC = 2
_NS = 16
_NBUF = 4
_LOOK = 2


def kernel(x):
    b, c, t, h, w = x.shape
    bc = b * c
    rows_out = bc * _NUM
    nw = _NC * _NS
    per = rows_out // nw
    hh = h // 2
    npc = per * 2

    xr = x.reshape(bc * t, h, w)
    mesh = plsc.VectorSubcoreMesh(
        core_axis_name="c", subcore_axis_name="s",
        num_cores=_NC, num_subcores=_NS,
    )

    @functools.partial(
        pl.kernel,
        out_type=jax.ShapeDtypeStruct((rows_out, h, w), x.dtype),
        mesh=mesh,
        scratch_types=[
            pltpu.VMEM((_NBUF, hh, w), jnp.float32),
            pltpu.SemaphoreType.DMA((_NBUF,)),
            pltpu.SemaphoreType.DMA((_NBUF,)),
        ],
    )
    def sc_gather(x_hbm, out_hbm, buf, sem_f, sem_s):
        wid = lax.axis_index("s") * _NC + lax.axis_index("c")
        base = wid * per

        def fetch(p):
            k, hf = divmod(p, 2)
            r = base + k
            g = r // _NUM
            j = r - g * _NUM
            src = g * t + (j * (t - 1)) // (_NUM - 1)
            return pltpu.make_async_copy(
                x_hbm.at[src, pl.ds(hf * hh, hh)],
                buf.at[p % _NBUF], sem_f.at[p % _NBUF])

        def store(p):
            k, hf = divmod(p, 2)
            return pltpu.make_async_copy(
                buf.at[p % _NBUF],
                out_hbm.at[base + k, pl.ds(hf * hh, hh)],
                sem_s.at[p % _NBUF])

        fetches = {}
        stores = {}
        for jj in range(npc + _LOOK):
            if jj < npc:
                if jj >= _NBUF:
                    stores[jj - _NBUF].wait()
                fetches[jj] = fetch(jj)
                fetches[jj].start()
            p = jj - _LOOK
            if 0 <= p < npc:
                fetches[p].wait()
                stores[p] = store(p)
                stores[p].start()
        for p in range(npc - _NBUF, npc):
            stores[p].wait()

    out = sc_gather(xr)
    return out.reshape(b, c, _NUM, h, w)


# speedup vs baseline: 1.0380x; 1.0380x over previous
import functools

import jax
import jax.numpy as jnp
from jax import lax
from jax.experimental import pallas as pl
from jax.experimental.pallas import tpu as pltpu
from jax.experimental.pallas import tpu_sc as plsc

_NUM = 16
_NC = 2
_NS = 16


def kernel(x):
    b, c, t, h, w = x.shape
    bc = b * c
    rows_out = bc * _NUM
    nw = _NC * _NS
    per = rows_out // nw

    xr = x.reshape(bc * t, h, w)
    mesh = plsc.VectorSubcoreMesh(
        core_axis_name="c", subcore_axis_name="s",
        num_cores=_NC, num_subcores=_NS,
    )

    @functools.partial(
        pl.kernel,
        out_type=jax.ShapeDtypeStruct((rows_out, h, w), x.dtype),
        mesh=mesh,
        scratch_types=[
            pltpu.VMEM_SHARED((_NS, 2, h, w), jnp.float32),
            pltpu.SemaphoreType.DMA,
            pltpu.SemaphoreType.DMA,
            pltpu.SemaphoreType.DMA,
        ],
    )
    def sc_gather(x_hbm, out_hbm, buf, sem_in, sem_out0, sem_out1):
        sid = lax.axis_index("s")
        wid = sid * _NC + lax.axis_index("c")
        base = wid * per
        sems_out = (sem_out0, sem_out1)

        def fetch(k):
            r = base + k
            g = r // _NUM
            j = r - g * _NUM
            src = g * t + (j * (t - 1)) // (_NUM - 1)
            return pltpu.make_async_copy(
                x_hbm.at[src], buf.at[sid, k % 2], sem_in)

        def store(k):
            return pltpu.make_async_copy(
                buf.at[sid, k % 2], out_hbm.at[base + k], sems_out[k % 2])

        stores = [None] * per
        fetch(0).start()
        for k in range(per):
            fetch(k).wait()
            stores[k] = store(k)
            stores[k].start()
            if k + 1 < per:
                if k >= 1:
                    stores[k - 1].wait()
                fetch(k + 1).start()
        stores[per - 1].wait()

    out = sc_gather(xr)
    return out.reshape(b, c, _NUM, h, w)
